# baseline (speedup 1.0000x reference)
import functools

import jax
import jax.numpy as jnp
from jax import lax
from jax.experimental import pallas as pl
from jax.experimental.pallas import tpu as pltpu
from jax.experimental.pallas import tpu_sc as plsc

_HALF = 32
_LANES = 16
_SLOT = 128
_CHUNK = 1024
_SLOTS = _CHUNK // _SLOT
_GRING = 4
_FRING = 2
_LEAD = 2
_NLAYERS = 3


def _sc_propagate(tab16, cat_tab, src2, dst2, vals, n_pad, per_tile):
    mesh = plsc.VectorSubcoreMesh(core_axis_name="c", subcore_axis_name="s")
    n_sub = mesh.num_subcores
    rows_per_tile = n_pad // n_sub
    n_sup = per_tile // _CHUNK
    out_f = jax.ShapeDtypeStruct((2 * n_pad, _HALF), jnp.float32)
    out_b = jax.ShapeDtypeStruct((2 * n_pad, _HALF // 2), jnp.int32)

    @functools.partial(
        pl.kernel,
        out_type=(out_f, out_f, out_f, out_b, out_b),
        mesh=mesh,
        compiler_params=pltpu.CompilerParams(use_tc_tiling_on_sc=False),
        scratch_types=[
            [pltpu.VMEM((_SLOT, _HALF // 2), jnp.int32)] * _GRING,
            [pltpu.VMEM((_SLOT, _HALF), jnp.float32)] * _FRING,
            pltpu.VMEM((_SLOT, _HALF // 2), jnp.int32),
            pltpu.VMEM((3, _SLOTS, _SLOT), jnp.int32),
            pltpu.VMEM((3, _SLOTS, _SLOT), jnp.int32),
            pltpu.VMEM((3, _CHUNK), jnp.float32),
            pltpu.VMEM((64, _HALF), jnp.float32),
            pltpu.VMEM_SHARED((n_pad, _HALF), jnp.float32),
            [pltpu.SemaphoreType.DMA] * _GRING,
            [pltpu.SemaphoreType.DMA] * _FRING,
            pltpu.SemaphoreType.DMA,
            pltpu.SemaphoreType.DMA,
        ],
    )
    def body(tab, srcr, dstr, valr, of1, of2, of3, ob1, ob2,
             gring, fring, wbuf, sidx3, didx3, vbuf3, zbuf, acc,
             gsems, ssems, psem, wsem):
        c = lax.axis_index("c")
        s = lax.axis_index("s")
        z16 = jnp.zeros((_LANES,), jnp.float32)

        def stage_descs(sup):
            b = lax.rem(sup, 3)
            row0 = pl.multiple_of(s * (per_tile // _SLOT) + sup * _SLOTS, 8)
            v0 = pl.multiple_of(s * per_tile + sup * _CHUNK, 8)
            return (
                pltpu.make_async_copy(srcr.at[c, pl.ds(row0, _SLOTS)], sidx3.at[b], psem),
                pltpu.make_async_copy(dstr.at[pl.ds(row0, _SLOTS)], didx3.at[b], psem),
                pltpu.make_async_copy(valr.at[pl.ds(v0, _CHUNK)], vbuf3.at[b], psem),
            )

        def gather_desc(b, slot, k):
            return pltpu.make_async_copy(
                tabref.at[sidx3.at[b, slot]], gring[k], gsems[k])

        def scatter_desc(b, slot, k):
            return pltpu.make_async_copy(
                fring[k], acc.at[didx3.at[b, slot]], ssems[k])

        def compute(b, slot, k, kf):
            gbuf = gring[k]
            fbuf = fring[kf]

            def grp(i, _):
                v = vbuf3[b, pl.ds(slot * _SLOT + i * _LANES, _LANES)]
                for j in range(_LANES):
                    e = i * _LANES + j
                    bc = jnp.take_along_axis(
                        v, jnp.full((_LANES,), j, jnp.int32), axis=0)
                    x = gbuf[e, :]
                    a_ = lax.bitcast_convert_type(
                        jnp.left_shift(x, 16), jnp.float32)
                    b_ = lax.bitcast_convert_type(
                        jnp.bitwise_and(x, jnp.int32(-65536)), jnp.float32)
                    fbuf[e, pl.ds(0, _LANES)] = a_ * bc
                    fbuf[e, pl.ds(_LANES, _LANES)] = b_ * bc
                return 0
            lax.fori_loop(0, _SLOT // _LANES, grp, 0)

        outs_f = (of1, of2, of3)
        outs_b = (ob1, ob2, None)
        for l in range(_NLAYERS):
            tabref = tab if l == 0 else outs_b[l - 1]

            for d_ in stage_descs(0):
                d_.start()

            def zfill(i, _):
                zbuf[i, pl.ds(0, _LANES)] = z16
                zbuf[i, pl.ds(_LANES, _LANES)] = z16
                return 0
            lax.fori_loop(0, 64, zfill, 0)
            zbase = s * rows_per_tile
            for j in range(rows_per_tile // 64):
                z0 = pl.multiple_of(zbase + j * 64, 8)
                pltpu.sync_copy(zbuf, acc.at[pl.ds(z0, 64)])
            zrem = rows_per_tile % 64
            if zrem:
                z0 = pl.multiple_of(zbase + (rows_per_tile // 64) * 64, 8)
                pltpu.sync_copy(zbuf.at[pl.ds(0, zrem)], acc.at[pl.ds(z0, zrem)])
            plsc.subcore_barrier()

            for d_ in stage_descs(0):
                d_.wait()
            for d_ in stage_descs(1):
                d_.start()
            for t in range(_LEAD):
                gather_desc(jnp.int32(0), t, t).start()

            def sup_body(sup, _):
                b = lax.rem(sup, 3)
                for t in range(_SLOTS):
                    k = t % _GRING
                    kf = t % _FRING
                    if t < _SLOTS - _LEAD:
                        gather_desc(b, t + _LEAD, (t + _LEAD) % _GRING).start()
                    gather_desc(b, t, k).wait()
                    if t < _FRING:
                        @pl.when(sup > 0)
                        def _(kf=kf):
                            scatter_desc(b, 0, kf).wait()
                    else:
                        scatter_desc(b, 0, kf).wait()
                    compute(b, t, k, kf)
                    scatter_desc(b, t, kf).start(add=True)

                @pl.when(sup < n_sup - 1)
                def _():
                    bn = lax.rem(sup + 1, 3)
                    for d_ in stage_descs(sup + 1):
                        d_.wait()

                    @pl.when(sup < n_sup - 2)
                    def _():
                        for d_ in stage_descs(sup + 2):
                            d_.start()
                    for t in range(_LEAD):
                        gather_desc(bn, t, t % _GRING).start()
                return 0
            lax.fori_loop(0, n_sup, sup_body, 0)

            for kf in range(_FRING):
                scatter_desc(jnp.int32(0), 0, kf).wait()
            plsc.subcore_barrier()

            r0_ = pl.multiple_of(s * rows_per_tile, 8)
            w0 = pl.multiple_of(c * n_pad + s * rows_per_tile, 8)
            pltpu.sync_copy(acc.at[pl.ds(r0_, rows_per_tile)],
                            outs_f[l].at[pl.ds(w0, rows_per_tile)])
            if l < _NLAYERS - 1:
                nwb = rows_per_tile // _SLOT
                def wb_chunk(base_rows, nrows):
                    ra = pl.multiple_of(r0_ + base_rows, 8)
                    wa = pl.multiple_of(w0 + base_rows, 8)
                    pltpu.sync_copy(acc.at[pl.ds(ra, nrows)],
                                    fring[0].at[pl.ds(0, nrows)])

                    rnd = jnp.int32(32768)
                    hmask = jnp.int32(-65536)

                    def prow(e, _):
                        ai = lax.bitcast_convert_type(
                            fring[0][e, pl.ds(0, _LANES)], jnp.int32)
                        bi = lax.bitcast_convert_type(
                            fring[0][e, pl.ds(_LANES, _LANES)], jnp.int32)
                        low = jnp.right_shift(
                            jnp.bitwise_and(ai + rnd, hmask), 16)
                        low = jnp.bitwise_and(low, jnp.int32(65535))
                        high = jnp.bitwise_and(bi + rnd, hmask)
                        wbuf[e, :] = jnp.bitwise_or(low, high)
                        return 0
                    lax.fori_loop(0, nrows, prow, 0)
                    pltpu.sync_copy(wbuf.at[pl.ds(0, nrows)],
                                    outs_b[l].at[pl.ds(wa, nrows)])

                def wb_loop(jj, _):
                    wb_chunk(jj * _SLOT, _SLOT)
                    return 0
                lax.fori_loop(0, nwb, wb_loop, 0)
                wrem = rows_per_tile % _SLOT
                if wrem:
                    wb_chunk(nwb * _SLOT, wrem)
            plsc.subcore_barrier()

    return body(tab16, src2, dst2, vals)


def _tc_finalize(cat_tab, o1, o2, o3, n_pad):
    blk = 544
    nblk = n_pad // blk
    half_off = n_pad // blk

    def fin(t0, t1, a0, a1, b0, b1, c0, c1, out):
        m0 = (t0[...] + a0[...] + b0[...] + c0[...]) * 0.25
        m1 = (t1[...] + a1[...] + b1[...] + c1[...]) * 0.25
        ns = jnp.sum(m0 * m0, axis=1, keepdims=True) + jnp.sum(m1 * m1, axis=1, keepdims=True)
        inv = 1.0 / jnp.maximum(jnp.sqrt(ns), 1e-12)
        out[:, :_HALF] = m0 * inv
        out[:, _HALF:] = m1 * inv

    spec0 = pl.BlockSpec((blk, _HALF), lambda i: (i, 0))
    spec1 = pl.BlockSpec((blk, _HALF), lambda i: (i + half_off, 0))
    return pl.pallas_call(
        fin,
        grid=(nblk,),
        in_specs=[spec0, spec1, spec0, spec1, spec0, spec1, spec0, spec1],
        out_specs=pl.BlockSpec((blk, 2 * _HALF), lambda i: (i, 0)),
        out_shape=jax.ShapeDtypeStruct((n_pad, 2 * _HALF), jnp.float32),
    )(cat_tab, cat_tab, o1, o1, o2, o2, o3, o3)


def kernel(user_emb, item_emb, edge_vals, edge_index):
    n_users = user_emb.shape[0]
    n_nodes = n_users + item_emb.shape[0]
    n_edges = edge_vals.shape[0]
    mesh = plsc.VectorSubcoreMesh(core_axis_name="c", subcore_axis_name="s")
    n_tiles = mesh.num_subcores
    n_pad = ((n_nodes + 8 * n_tiles - 1) // (8 * n_tiles)) * (8 * n_tiles)
    e_pad = ((n_edges + n_tiles * _CHUNK - 1) // (n_tiles * _CHUNK)) * (n_tiles * _CHUNK)
    per_tile = e_pad // n_tiles

    all_emb = jnp.concatenate([user_emb, item_emb], axis=0).astype(jnp.float32)
    all_emb = jnp.pad(all_emb, ((0, n_pad - n_nodes), (0, 0)))
    cat_tab = jnp.concatenate([all_emb[:, :_HALF], all_emb[:, _HALF:]], axis=0)
    a16 = all_emb.astype(jnp.bfloat16)
    h0 = jnp.stack([a16[:, 0:16], a16[:, 16:32]], axis=2)
    h1 = jnp.stack([a16[:, 32:48], a16[:, 48:64]], axis=2)
    tab16 = jax.lax.bitcast_convert_type(
        jnp.concatenate([h0, h1], axis=0), jnp.int32)

    src = edge_index[0].astype(jnp.int32)
    dst = edge_index[1].astype(jnp.int32)
    vals = edge_vals.astype(jnp.float32)
    pad = e_pad - n_edges
    src = jnp.concatenate([src, jnp.zeros((pad,), jnp.int32)])
    dst = jnp.concatenate([dst, jnp.zeros((pad,), jnp.int32)])
    vals = jnp.concatenate([vals, jnp.zeros((pad,), jnp.float32)])
    src2 = jnp.stack([src, src + n_pad]).reshape(2, e_pad // _SLOT, _SLOT)
    dst2 = dst.reshape(e_pad // _SLOT, _SLOT)

    o1, o2, o3, _, _ = _sc_propagate(
        tab16, cat_tab, src2, dst2, vals, n_pad, per_tile)
    res = _tc_finalize(cat_tab, o1, o2, o3, n_pad)
    return res[:n_users], res[n_users:n_nodes]


# speedup vs baseline: 1.4083x; 1.4083x over previous
import functools

import jax
import jax.numpy as jnp
from jax import lax
from jax.experimental import pallas as pl
from jax.experimental.pallas import tpu as pltpu
from jax.experimental.pallas import tpu_sc as plsc

_HALF = 32
_LANES = 16
_SLOT = 128
_CHUNK = 1024
_SLOTS = _CHUNK // _SLOT
_RING = 4
_LEAD = 2
_NLAYERS = 3


def _sc_propagate(cat_tab, src2, dst2, vals, n_pad, per_tile):
    mesh = plsc.VectorSubcoreMesh(core_axis_name="c", subcore_axis_name="s")
    n_sub = mesh.num_subcores
    rows_per_tile = n_pad // n_sub
    n_sup = per_tile // _CHUNK
    out_sds = jax.ShapeDtypeStruct((2 * n_pad, _HALF), jnp.float32)

    @functools.partial(
        pl.kernel,
        out_type=(out_sds, out_sds, out_sds),
        mesh=mesh,
        compiler_params=pltpu.CompilerParams(use_tc_tiling_on_sc=False),
        scratch_types=[
            [pltpu.VMEM((_SLOT, _HALF), jnp.float32)] * _RING,
            pltpu.VMEM((3, _SLOTS, _SLOT), jnp.int32),
            pltpu.VMEM((3, _SLOTS, _SLOT), jnp.int32),
            pltpu.VMEM((3, _CHUNK), jnp.float32),
            pltpu.VMEM((64, _HALF), jnp.float32),
            pltpu.VMEM_SHARED((n_pad, _HALF), jnp.float32),
            [pltpu.SemaphoreType.DMA] * _RING,
            [pltpu.SemaphoreType.DMA] * _RING,
            pltpu.SemaphoreType.DMA,
        ],
    )
    def body(tab, srcr, dstr, valr, out1, out2, out3,
             ring, sidx3, didx3, vbuf3, zbuf, acc, gsems, ssems, psem):
        c = lax.axis_index("c")
        s = lax.axis_index("s")
        z16 = jnp.zeros((_LANES,), jnp.float32)

        def stage_descs(sup):
            b = lax.rem(sup, 3)
            row0 = pl.multiple_of(s * (per_tile // _SLOT) + sup * _SLOTS, 8)
            v0 = pl.multiple_of(s * per_tile + sup * _CHUNK, 8)
            return (
                pltpu.make_async_copy(srcr.at[c, pl.ds(row0, _SLOTS)], sidx3.at[b], psem),
                pltpu.make_async_copy(dstr.at[pl.ds(row0, _SLOTS)], didx3.at[b], psem),
                pltpu.make_async_copy(valr.at[pl.ds(v0, _CHUNK)], vbuf3.at[b], psem),
            )

        def gather_desc(b, slot, k):
            return pltpu.make_async_copy(
                tabref.at[sidx3.at[b, slot]], ring[k], gsems[k])

        def scatter_desc(b, slot, k):
            return pltpu.make_async_copy(
                ring[k], acc.at[didx3.at[b, slot]], ssems[k])

        def compute(b, slot, k):
            rbuf = ring[k]

            def grp(i, _):
                v = vbuf3[b, pl.ds(slot * _SLOT + i * _LANES, _LANES)]
                for j in range(_LANES):
                    e = i * _LANES + j
                    bc = jnp.take_along_axis(
                        v, jnp.full((_LANES,), j, jnp.int32), axis=0)
                    rbuf[e, pl.ds(0, _LANES)] = rbuf[e, pl.ds(0, _LANES)] * bc
                    rbuf[e, pl.ds(_LANES, _LANES)] = (
                        rbuf[e, pl.ds(_LANES, _LANES)] * bc)
                return 0
            lax.fori_loop(0, _SLOT // _LANES, grp, 0)

        outs = (out1, out2, out3)
        for l in range(_NLAYERS):
            tabref = tab if l == 0 else outs[l - 1]

            for d_ in stage_descs(0):
                d_.start()

            def zfill(i, _):
                zbuf[i, pl.ds(0, _LANES)] = z16
                zbuf[i, pl.ds(_LANES, _LANES)] = z16
                return 0
            lax.fori_loop(0, 64, zfill, 0)
            zbase = s * rows_per_tile
            for j in range(rows_per_tile // 64):
                z0 = pl.multiple_of(zbase + j * 64, 8)
                pltpu.sync_copy(zbuf, acc.at[pl.ds(z0, 64)])
            zrem = rows_per_tile % 64
            if zrem:
                z0 = pl.multiple_of(zbase + (rows_per_tile // 64) * 64, 8)
                pltpu.sync_copy(zbuf.at[pl.ds(0, zrem)], acc.at[pl.ds(z0, zrem)])
            plsc.subcore_barrier()

            for d_ in stage_descs(0):
                d_.wait()
            for d_ in stage_descs(1):
                d_.start()
            for t in range(_LEAD):
                gather_desc(jnp.int32(0), t, t).start()

            def sup_body(sup, _):
                b = lax.rem(sup, 3)
                for t in range(_SLOTS):
                    k = t % _RING
                    if t < _SLOTS - _LEAD:
                        kk = (t + _LEAD) % _RING
                        if t < _LEAD:
                            @pl.when(sup > 0)
                            def _(kk=kk):
                                scatter_desc(b, 0, kk).wait()
                        else:
                            scatter_desc(b, 0, kk).wait()
                        gather_desc(b, t + _LEAD, kk).start()
                    gather_desc(b, t, k).wait()
                    compute(b, t, k)
                    scatter_desc(b, t, k).start(add=True)

                @pl.when(sup < n_sup - 1)
                def _():
                    bn = lax.rem(sup + 1, 3)
                    for d_ in stage_descs(sup + 1):
                        d_.wait()

                    @pl.when(sup < n_sup - 2)
                    def _():
                        for d_ in stage_descs(sup + 2):
                            d_.start()
                    for t in range(_LEAD):
                        scatter_desc(bn, 0, t % _RING).wait()
                        gather_desc(bn, t, t % _RING).start()
                return 0
            lax.fori_loop(0, n_sup, sup_body, 0)

            for k in range(_RING):
                scatter_desc(jnp.int32(0), 0, k).wait()
            plsc.subcore_barrier()

            r0_ = pl.multiple_of(s * rows_per_tile, 8)
            w0 = pl.multiple_of(c * n_pad + s * rows_per_tile, 8)
            pltpu.sync_copy(acc.at[pl.ds(r0_, rows_per_tile)],
                            outs[l].at[pl.ds(w0, rows_per_tile)])
            plsc.subcore_barrier()

    return body(cat_tab, src2, dst2, vals)


def _tc_finalize(cat_tab, o1, o2, o3, n_pad):
    blk = 544
    nblk = n_pad // blk
    half_off = n_pad // blk

    def fin(t0, t1, a0, a1, b0, b1, c0, c1, out):
        m0 = (t0[...] + a0[...] + b0[...] + c0[...]) * 0.25
        m1 = (t1[...] + a1[...] + b1[...] + c1[...]) * 0.25
        ns = jnp.sum(m0 * m0, axis=1, keepdims=True) + jnp.sum(m1 * m1, axis=1, keepdims=True)
        inv = 1.0 / jnp.maximum(jnp.sqrt(ns), 1e-12)
        out[:, :_HALF] = m0 * inv
        out[:, _HALF:] = m1 * inv

    spec0 = pl.BlockSpec((blk, _HALF), lambda i: (i, 0))
    spec1 = pl.BlockSpec((blk, _HALF), lambda i: (i + half_off, 0))
    return pl.pallas_call(
        fin,
        grid=(nblk,),
        in_specs=[spec0, spec1, spec0, spec1, spec0, spec1, spec0, spec1],
        out_specs=pl.BlockSpec((blk, 2 * _HALF), lambda i: (i, 0)),
        out_shape=jax.ShapeDtypeStruct((n_pad, 2 * _HALF), jnp.float32),
    )(cat_tab, cat_tab, o1, o1, o2, o2, o3, o3)


def kernel(user_emb, item_emb, edge_vals, edge_index):
    n_users = user_emb.shape[0]
    n_nodes = n_users + item_emb.shape[0]
    n_edges = edge_vals.shape[0]
    mesh = plsc.VectorSubcoreMesh(core_axis_name="c", subcore_axis_name="s")
    n_tiles = mesh.num_subcores
    n_pad = ((n_nodes + 8 * n_tiles - 1) // (8 * n_tiles)) * (8 * n_tiles)
    e_pad = ((n_edges + n_tiles * _CHUNK - 1) // (n_tiles * _CHUNK)) * (n_tiles * _CHUNK)
    per_tile = e_pad // n_tiles

    all_emb = jnp.concatenate([user_emb, item_emb], axis=0).astype(jnp.float32)
    all_emb = jnp.pad(all_emb, ((0, n_pad - n_nodes), (0, 0)))
    cat_tab = jnp.concatenate([all_emb[:, :_HALF], all_emb[:, _HALF:]], axis=0)

    src = edge_index[0].astype(jnp.int32)
    dst = edge_index[1].astype(jnp.int32)
    vals = edge_vals.astype(jnp.float32)
    pad = e_pad - n_edges
    src = jnp.concatenate([src, jnp.zeros((pad,), jnp.int32)])
    dst = jnp.concatenate([dst, jnp.zeros((pad,), jnp.int32)])
    vals = jnp.concatenate([vals, jnp.zeros((pad,), jnp.float32)])
    src2 = jnp.stack([src, src + n_pad]).reshape(2, e_pad // _SLOT, _SLOT)
    dst2 = dst.reshape(e_pad // _SLOT, _SLOT)

    o1, o2, o3 = _sc_propagate(cat_tab, src2, dst2, vals, n_pad, per_tile)
    res = _tc_finalize(cat_tab, o1, o2, o3, n_pad)
    return res[:n_users], res[n_users:n_nodes]
